# baseline (speedup 1.0000x reference)
import functools

import jax
import jax.numpy as jnp
from jax import lax
from jax.experimental import pallas as pl
from jax.experimental.pallas import tpu as pltpu
from jax.experimental.pallas import tpu_sc as plsc

_L = 16


def _make_sc_kernel(B, V, D, NC, NS):
    NW = NC * NS
    b_per_w = B // NW

    mesh = plsc.VectorSubcoreMesh(core_axis_name="c", subcore_axis_name="s")

    @functools.partial(
        pl.kernel,
        mesh=mesh,
        out_type=jax.ShapeDtypeStruct((B,), jnp.float32),
        compiler_params=pltpu.CompilerParams(use_tc_tiling_on_sc=False),
        scratch_types=[
            pltpu.VMEM((b_per_w,), jnp.int32),
            pltpu.VMEM((b_per_w,), jnp.int32),
            pltpu.VMEM((b_per_w, D), jnp.float32),
            pltpu.VMEM((b_per_w, D), jnp.float32),
            pltpu.VMEM((b_per_w,), jnp.float32),
            pltpu.SemaphoreType.DMA,
            pltpu.SemaphoreType.DMA,
        ],
    )
    def sc_k(ti_hbm, ci_hbm, tbl_hbm, out_hbm,
             ti_v, ci_v, t_rows, c_rows, out_v, sem_t, sem_c):
        wid = lax.axis_index("s") * NC + lax.axis_index("c")
        base = wid * b_per_w
        pltpu.sync_copy(ti_hbm.at[pl.ds(base, b_per_w)], ti_v)
        pltpu.sync_copy(ci_hbm.at[pl.ds(base, b_per_w)], ci_v)
        cp_t = pltpu.make_async_copy(tbl_hbm.at[ti_v], t_rows, sem_t)
        cp_c = pltpu.make_async_copy(tbl_hbm.at[ci_v], c_rows, sem_c)
        cp_t.start()
        cp_c.start()
        cp_t.wait()
        cp_c.wait()

        nvec = D // _L
        lane = lax.iota(jnp.int32, _L)
        perms = [lane ^ s for s in (8, 4, 2, 1)]

        def grp_body(g, carry):
            res = jnp.zeros((_L,), jnp.float32)
            for u in range(_L):
                i = g * _L + u
                acc = (t_rows[i, pl.ds(0, _L)] * c_rows[i, pl.ds(0, _L)])
                for k in range(1, nvec):
                    acc = acc + (t_rows[i, pl.ds(k * _L, _L)]
                                 * c_rows[i, pl.ds(k * _L, _L)])
                for p in perms:
                    acc = acc + acc.at[p].get(mode="promise_in_bounds")
                res = jnp.where(lane == u, acc, res)
            out_v[pl.ds(g * _L, _L)] = 1.0 / (1.0 + jnp.exp(-res))
            return carry

        lax.fori_loop(0, b_per_w // _L, grp_body, 0)
        pltpu.sync_copy(out_v, out_hbm.at[pl.ds(base, b_per_w)])

    return sc_k


def kernel(pair_items, table):
    B = pair_items.shape[1]
    V, D = table.shape
    info = plsc.get_sparse_core_info()
    sc_k = _make_sc_kernel(B, V, D, info.num_cores, info.num_subcores)
    return sc_k(pair_items[0], pair_items[1], table)


# speedup vs baseline: 1.6930x; 1.6930x over previous
import functools

import jax
import jax.numpy as jnp
from jax import lax
from jax.experimental import pallas as pl
from jax.experimental.pallas import tpu as pltpu
from jax.experimental.pallas import tpu_sc as plsc

_L = 16


def _make_sc_kernel(B, V, D, NC, NS):
    NW = NC * NS
    b_per_w = B // NW
    CH = 256
    n_ch = b_per_w // CH
    nvec = D // _L

    mesh = plsc.VectorSubcoreMesh(core_axis_name="c", subcore_axis_name="s")

    @functools.partial(
        pl.kernel,
        mesh=mesh,
        out_type=jax.ShapeDtypeStruct((B,), jnp.float32),
        scratch_types=[
            pltpu.VMEM((b_per_w,), jnp.int32),
            pltpu.VMEM((b_per_w,), jnp.int32),
            pltpu.VMEM((CH, D), jnp.float32),
            pltpu.VMEM((CH, D), jnp.float32),
            pltpu.VMEM((b_per_w,), jnp.float32),
            pltpu.SemaphoreType.DMA,
            pltpu.SemaphoreType.DMA,
        ],
    )
    def sc_k(ti_hbm, ci_hbm, tbl_hbm, out_hbm,
             ti_v, ci_v, t_rows, c_rows, out_v, sem_t, sem_c):
        wid = lax.axis_index("s") * NC + lax.axis_index("c")
        base = wid * b_per_w
        pltpu.sync_copy(ti_hbm.at[pl.ds(base, b_per_w)], ti_v)
        pltpu.sync_copy(ci_hbm.at[pl.ds(base, b_per_w)], ci_v)

        lane = lax.iota(jnp.int32, _L)
        perms = [lane ^ s for s in (8, 4, 2, 1)]

        def chunk_body(ch, carry):
            off = ch * CH

            def fire_body(g, carry2):
                tvec = ti_v[pl.ds(off + g * _L, _L)]
                cvec = ci_v[pl.ds(off + g * _L, _L)]
                for u in range(_L):
                    pltpu.make_async_copy(
                        tbl_hbm.at[pl.ds(tvec[u], 1)],
                        t_rows.at[pl.ds(g * _L + u, 1)], sem_t).start()
                    pltpu.make_async_copy(
                        tbl_hbm.at[pl.ds(cvec[u], 1)],
                        c_rows.at[pl.ds(g * _L + u, 1)], sem_c).start()
                return carry2

            lax.fori_loop(0, CH // _L, fire_body, 0)
            pltpu.make_async_copy(
                tbl_hbm.at[pl.ds(0, CH)], t_rows, sem_t).wait()
            pltpu.make_async_copy(
                tbl_hbm.at[pl.ds(0, CH)], c_rows, sem_c).wait()

            def grp_body(g, carry2):
                res = jnp.zeros((_L,), jnp.float32)
                for u in range(_L):
                    i = g * _L + u
                    acc = (t_rows[i, pl.ds(0, _L)] * c_rows[i, pl.ds(0, _L)])
                    for k in range(1, nvec):
                        acc = acc + (t_rows[i, pl.ds(k * _L, _L)]
                                     * c_rows[i, pl.ds(k * _L, _L)])
                    for p in perms:
                        acc = acc + acc.at[p].get(mode="promise_in_bounds")
                    res = jnp.where(lane == u, acc, res)
                out_v[pl.ds(off + g * _L, _L)] = 1.0 / (1.0 + jnp.exp(-res))
                return carry2

            lax.fori_loop(0, CH // _L, grp_body, 0)
            return carry

        lax.fori_loop(0, n_ch, chunk_body, 0)
        pltpu.sync_copy(out_v, out_hbm.at[pl.ds(base, b_per_w)])

    return sc_k


def kernel(pair_items, table):
    B = pair_items.shape[1]
    V, D = table.shape
    info = plsc.get_sparse_core_info()
    sc_k = _make_sc_kernel(B, V, D, info.num_cores, info.num_subcores)
    return sc_k(pair_items[0], pair_items[1], table)
